# baseline (speedup 1.0000x reference)
import jax
import jax.numpy as jnp
from jax.experimental import pallas as pl
from jax.experimental.pallas import tpu as pltpu

N = 100000
STATIC_DIM = 128
DYNAMIC_DIM = 128
OUT_DIM = STATIC_DIM + DYNAMIC_DIM
N_CHUNKS = 10
ROWS = N // N_CHUNKS


def _dma_body(a_ref, b_ref, o_ref, sems):
    copies = []
    for c in range(N_CHUNKS):
        r = pl.ds(c * ROWS, ROWS)
        copies.append(pltpu.make_async_copy(
            a_ref.at[r, :], o_ref.at[r, pl.ds(0, STATIC_DIM)], sems.at[2 * c]))
        copies.append(pltpu.make_async_copy(
            b_ref.at[r, :], o_ref.at[r, pl.ds(STATIC_DIM, DYNAMIC_DIM)],
            sems.at[2 * c + 1]))
    for cp in copies:
        cp.start()
    for cp in copies:
        cp.wait()


def kernel(static_emb, dynamic_emb):
    return pl.pallas_call(
        _dma_body,
        in_specs=[
            pl.BlockSpec(memory_space=pltpu.MemorySpace.HBM),
            pl.BlockSpec(memory_space=pltpu.MemorySpace.HBM),
        ],
        out_specs=pl.BlockSpec(memory_space=pltpu.MemorySpace.HBM),
        out_shape=jax.ShapeDtypeStruct((N, OUT_DIM), jnp.float32),
        scratch_shapes=[pltpu.SemaphoreType.DMA((2 * N_CHUNKS,))],
    )(static_emb, dynamic_emb)


# speedup vs baseline: 33.0202x; 33.0202x over previous
import jax
import jax.numpy as jnp
from jax.experimental import pallas as pl
from jax.experimental.pallas import tpu as pltpu

N = 100000
STATIC_DIM = 128
DYNAMIC_DIM = 128
OUT_DIM = STATIC_DIM + DYNAMIC_DIM
BLOCK_N = 2000


def _body(a_ref, b_ref, o_ref, sem_a, sem_b):
    base = pl.multiple_of(pl.program_id(0) * BLOCK_N, 8)
    rows = pl.ds(base, BLOCK_N)
    ca = pltpu.make_async_copy(
        a_ref, o_ref.at[rows, pl.ds(0, STATIC_DIM)], sem_a)
    cb = pltpu.make_async_copy(
        b_ref, o_ref.at[rows, pl.ds(STATIC_DIM, DYNAMIC_DIM)], sem_b)
    ca.start()
    cb.start()
    ca.wait()
    cb.wait()


def kernel(static_emb, dynamic_emb):
    return pl.pallas_call(
        _body,
        grid=(N // BLOCK_N,),
        in_specs=[
            pl.BlockSpec((BLOCK_N, STATIC_DIM), lambda i: (i, 0)),
            pl.BlockSpec((BLOCK_N, DYNAMIC_DIM), lambda i: (i, 0)),
        ],
        out_specs=pl.BlockSpec(memory_space=pltpu.MemorySpace.HBM),
        out_shape=jax.ShapeDtypeStruct((N, OUT_DIM), jnp.float32),
        scratch_shapes=[pltpu.SemaphoreType.DMA, pltpu.SemaphoreType.DMA],
    )(static_emb, dynamic_emb)


# speedup vs baseline: 33.0856x; 1.0020x over previous
import jax
import jax.numpy as jnp
from jax.experimental import pallas as pl
from jax.experimental.pallas import tpu as pltpu

N = 100000
STATIC_DIM = 128
DYNAMIC_DIM = 128
OUT_DIM = STATIC_DIM + DYNAMIC_DIM
BLOCK_N = 4000


def _body(a_ref, b_ref, o_ref, sem_a, sem_b):
    base = pl.multiple_of(pl.program_id(0) * BLOCK_N, 8)
    rows = pl.ds(base, BLOCK_N)
    ca = pltpu.make_async_copy(
        a_ref.at[rows, :], o_ref.at[:, pl.ds(0, STATIC_DIM)], sem_a)
    cb = pltpu.make_async_copy(
        b_ref.at[rows, :], o_ref.at[:, pl.ds(STATIC_DIM, DYNAMIC_DIM)], sem_b)
    ca.start()
    cb.start()
    ca.wait()
    cb.wait()


def kernel(static_emb, dynamic_emb):
    return pl.pallas_call(
        _body,
        grid=(N // BLOCK_N,),
        in_specs=[
            pl.BlockSpec(memory_space=pltpu.MemorySpace.HBM),
            pl.BlockSpec(memory_space=pltpu.MemorySpace.HBM),
        ],
        out_specs=pl.BlockSpec((BLOCK_N, OUT_DIM), lambda i: (i, 0)),
        out_shape=jax.ShapeDtypeStruct((N, OUT_DIM), jnp.float32),
        scratch_shapes=[pltpu.SemaphoreType.DMA, pltpu.SemaphoreType.DMA],
    )(static_emb, dynamic_emb)


# speedup vs baseline: 47.6215x; 1.4393x over previous
import jax
import jax.numpy as jnp
from jax.experimental import pallas as pl
from jax.experimental.pallas import tpu as pltpu

N = 100000
STATIC_DIM = 128
DYNAMIC_DIM = 128
OUT_DIM = STATIC_DIM + DYNAMIC_DIM
BLOCK_N = 4000


def _body(a_ref, b_ref, o_ref, sem_a, sem_b):
    ca = pltpu.make_async_copy(a_ref, o_ref.at[:, pl.ds(0, STATIC_DIM)], sem_a)
    cb = pltpu.make_async_copy(
        b_ref, o_ref.at[:, pl.ds(STATIC_DIM, DYNAMIC_DIM)], sem_b)
    ca.start()
    cb.start()
    ca.wait()
    cb.wait()


def kernel(static_emb, dynamic_emb):
    return pl.pallas_call(
        _body,
        grid=(N // BLOCK_N,),
        in_specs=[
            pl.BlockSpec((BLOCK_N, STATIC_DIM), lambda i: (i, 0)),
            pl.BlockSpec((BLOCK_N, DYNAMIC_DIM), lambda i: (i, 0)),
        ],
        out_specs=pl.BlockSpec((BLOCK_N, OUT_DIM), lambda i: (i, 0)),
        out_shape=jax.ShapeDtypeStruct((N, OUT_DIM), jnp.float32),
        scratch_shapes=[pltpu.SemaphoreType.DMA, pltpu.SemaphoreType.DMA],
    )(static_emb, dynamic_emb)
